# Initial kernel scaffold; baseline (speedup 1.0000x reference)
import jax
import jax.numpy as jnp
from jax.experimental import pallas as pl


def kernel(question_embedding, query_from_type, user_dense, gender, platform, age, location, user_idx, recent_clicked_note_idxs, note_embedding, note_dense, note_type, taxonomy1_id, taxonomy2_id, taxonomy3_id, note_idx, emb_gender, emb_platform, emb_age, emb_location, emb_user_idx, emb_note_type, emb_tax1, emb_tax2, emb_tax3, emb_note_idx, cross_w, cross_b, cross_g, cross_beta, W1, b1, g1, beta1, W2, b2, g2, beta2, Wout, bout, gout, betaout):
    raise NotImplementedError("write your pallas kernel here")


# speedup vs baseline: 5.8802x; 5.8802x over previous
import functools

import jax
import jax.numpy as jnp
from jax import lax
from jax.experimental import pallas as pl
from jax.experimental.pallas import tpu as pltpu
from jax.experimental.pallas import tpu_sc as plsc

_B = 16384
_H = 200
_D = 1910
_HID = 256
_EPS = 1e-5
_NW = 32
_RW = _B // _NW
_GROUP = 8
_F32 = jnp.float32


def _sc_body(tab, hist_flat, note2d, qft2d, g2d, p2d, a2d, l2d, u2d, nt2d,
             t12d, t22d, t32d, eye16, tg, tp, ta, tl, tu, tnt, tt1, tt2, tt3,
             o_hist, o_note, o_qft, o_g, o_p, o_a, o_l, o_u, o_nt, o_t1,
             o_t2, o_t3, sidx, r8, r16, r32, r64, hidx, hrows, hacc,
             gsem, hsem):
    wid = lax.axis_index("s") * 2 + lax.axis_index("c")

    small = [
        (note2d, tab, o_note, r32),
        (qft2d, eye16, o_qft, r16),
        (g2d, tg, o_g, r8),
        (p2d, tp, o_p, r8),
        (a2d, ta, o_a, r16),
        (l2d, tl, o_l, r32),
        (u2d, tu, o_u, r32),
        (nt2d, tnt, o_nt, r8),
        (t12d, tt1, o_t1, r16),
        (t22d, tt2, o_t2, r32),
        (t32d, tt3, o_t3, r64),
    ]
    for idx2d_, tab_, out_, buf_ in small:
        def chunk(c, _, idx2d=idx2d_, tabref=tab_, outref=out_, buf=buf_):
            row = wid * 4 + c
            pltpu.sync_copy(idx2d.at[pl.ds(row, 1)], sidx)
            pltpu.async_copy(tabref.at[sidx.at[0]], buf, gsem).wait()
            pltpu.sync_copy(buf, outref.at[pl.ds(row * 128, 128)])
            return 0
        lax.fori_loop(0, 4, chunk, 0)

    def group(gi, _):
        row0 = wid * _RW + gi * _GROUP
        flat0 = row0 * _H
        pltpu.sync_copy(hist_flat.at[pl.ds(flat0, _GROUP * _H)], hidx)
        cps = [pltpu.async_copy(tab.at[hidx.at[pl.ds(j * 128, 128)]],
                                hrows.at[pl.ds(j * 128, 128)], hsem)
               for j in range(12)]
        cps.append(pltpu.async_copy(tab.at[hidx.at[pl.ds(1536, 64)]],
                                    hrows.at[pl.ds(1536, 64)], hsem))
        for cp in cps:
            cp.wait()

        def row_acc(r, _):
            base = r * _H
            zero = jnp.zeros((16,), _F32)

            def step(k, carry):
                a0, a1, b0, b1 = carry
                o = base + k * 8
                for u in range(8):
                    lo = hrows[o + u, pl.ds(0, 16)]
                    hi = hrows[o + u, pl.ds(16, 16)]
                    if u % 2 == 0:
                        a0 = a0 + lo
                        a1 = a1 + hi
                    else:
                        b0 = b0 + lo
                        b1 = b1 + hi
                return (a0, a1, b0, b1)

            a0, a1, b0, b1 = lax.fori_loop(0, _H // 8, step,
                                           (zero, zero, zero, zero))
            hacc[r, pl.ds(0, 16)] = (a0 + b0) * (1.0 / _H)
            hacc[r, pl.ds(16, 16)] = (a1 + b1) * (1.0 / _H)
            return 0

        lax.fori_loop(0, _GROUP, row_acc, 0)
        pltpu.sync_copy(hacc, o_hist.at[pl.ds(row0, _GROUP)])
        return 0

    lax.fori_loop(0, _RW // _GROUP, group, 0)


def _sc_gather_all(tab, hist_flat, note2d, qft2d, g2d, p2d, a2d, l2d, u2d,
                   nt2d, t12d, t22d, t32d, eye16, tg, tp, ta, tl, tu, tnt,
                   tt1, tt2, tt3):
    mesh = plsc.VectorSubcoreMesh(core_axis_name="c", subcore_axis_name="s")
    out_type = [
        jax.ShapeDtypeStruct((_B, 32), _F32),
        jax.ShapeDtypeStruct((_B, 32), _F32),
        jax.ShapeDtypeStruct((_B, 16), _F32),
        jax.ShapeDtypeStruct((_B, 8), _F32),
        jax.ShapeDtypeStruct((_B, 8), _F32),
        jax.ShapeDtypeStruct((_B, 16), _F32),
        jax.ShapeDtypeStruct((_B, 32), _F32),
        jax.ShapeDtypeStruct((_B, 32), _F32),
        jax.ShapeDtypeStruct((_B, 8), _F32),
        jax.ShapeDtypeStruct((_B, 16), _F32),
        jax.ShapeDtypeStruct((_B, 32), _F32),
        jax.ShapeDtypeStruct((_B, 64), _F32),
    ]
    scratch_types = [
        pltpu.VMEM((1, 128), jnp.int32),
        pltpu.VMEM((128, 8), _F32),
        pltpu.VMEM((128, 16), _F32),
        pltpu.VMEM((128, 32), _F32),
        pltpu.VMEM((128, 64), _F32),
        pltpu.VMEM((_GROUP * _H,), jnp.int32),
        pltpu.VMEM((_GROUP * _H, 32), _F32),
        pltpu.VMEM((_GROUP, 32), _F32),
        pltpu.SemaphoreType.DMA,
        pltpu.SemaphoreType.DMA,
    ]
    fn = functools.partial(
        pl.kernel, mesh=mesh, out_type=out_type,
        scratch_types=scratch_types,
        compiler_params=pltpu.CompilerParams(use_tc_tiling_on_sc=False),
    )(_sc_body)
    return fn(tab, hist_flat, note2d, qft2d, g2d, p2d, a2d, l2d, u2d, nt2d,
              t12d, t22d, t32d, eye16, tg, tp, ta, tl, tu, tnt, tt1, tt2, tt3)


_TILE = 512
_NT = _B // _TILE


def _cp():
    return pltpu.CompilerParams(dimension_semantics=("arbitrary",))


def _k0_body(x0_ref, w1_ref, w0_ref, y1_ref, mom_ref, ys_ref, xw0_ref):
    x0 = x0_ref[...]
    xw0 = jnp.dot(x0, w0_ref[...], preferred_element_type=_F32)
    y1 = jnp.dot(x0, w1_ref[...], preferred_element_type=_F32)
    y1_ref[...] = y1
    x2 = x0 * x0
    m1 = jnp.sum(x0, 0, keepdims=True)
    m2 = jnp.sum(x0 * xw0, 0, keepdims=True)
    p11 = jnp.sum(x2, 0, keepdims=True)
    p12 = jnp.sum(x2 * xw0, 0, keepdims=True)
    p22 = jnp.sum(x2 * (xw0 * xw0), 0, keepdims=True)
    mom_ref[...] = jnp.concatenate([m1, m2, p11, p12, p22], 0)[None]
    ys_ref[...] = jnp.concatenate([jnp.sum(y1, 0, keepdims=True),
                                   jnp.sum(y1 * y1, 0, keepdims=True)],
                                  0)[None]
    xw0_ref[...] = xw0


def _k0(x0, W1, w0):
    return pl.pallas_call(
        _k0_body,
        grid=(_NT,),
        in_specs=[
            pl.BlockSpec((_TILE, _D), lambda i: (i, 0)),
            pl.BlockSpec((_D, _HID), lambda i: (0, 0)),
            pl.BlockSpec((_D, 1), lambda i: (0, 0)),
        ],
        out_specs=[
            pl.BlockSpec((_TILE, _HID), lambda i: (i, 0)),
            pl.BlockSpec((1, 5, _D), lambda i: (i, 0, 0)),
            pl.BlockSpec((1, 2, _HID), lambda i: (i, 0, 0)),
            pl.BlockSpec((_TILE, 1), lambda i: (i, 0)),
        ],
        out_shape=[
            jax.ShapeDtypeStruct((_B, _HID), _F32),
            jax.ShapeDtypeStruct((_NT, 5, _D), _F32),
            jax.ShapeDtypeStruct((_NT, 2, _HID), _F32),
            jax.ShapeDtypeStruct((_B, 1), _F32),
        ],
        compiler_params=_cp(),
    )(x0, W1, w0)


def _k1_body(x0_ref, y1_ref, xw0_ref, v_ref, c_ref, gb_ref, bb_ref, w2_ref,
             y2_ref, mom_ref, ys_ref, xw1_ref):
    x0 = x0_ref[...]
    xw0 = xw0_ref[...]
    e = jnp.dot(x0, v_ref[...], preferred_element_type=_F32)
    xw1 = e * (1.0 + xw0) + c_ref[0, 0]
    h1 = jnp.maximum(y1_ref[...] * gb_ref[...] + bb_ref[...], 0.0)
    y2 = jnp.dot(h1, w2_ref[...], preferred_element_type=_F32)
    y2_ref[...] = y2
    x2 = x0 * x0
    m3 = jnp.sum(x0 * xw1, 0, keepdims=True)
    p13 = jnp.sum(x2 * xw1, 0, keepdims=True)
    p23 = jnp.sum(x2 * (xw0 * xw1), 0, keepdims=True)
    p33 = jnp.sum(x2 * (xw1 * xw1), 0, keepdims=True)
    mom_ref[...] = jnp.concatenate([m3, p13, p23, p33], 0)[None]
    ys_ref[...] = jnp.concatenate([jnp.sum(y2, 0, keepdims=True),
                                   jnp.sum(y2 * y2, 0, keepdims=True)],
                                  0)[None]
    xw1_ref[...] = xw1


def _k1(x0, Y1p, xw0, v, c, gb, bb, W2):
    return pl.pallas_call(
        _k1_body,
        grid=(_NT,),
        in_specs=[
            pl.BlockSpec((_TILE, _D), lambda i: (i, 0)),
            pl.BlockSpec((_TILE, _HID), lambda i: (i, 0)),
            pl.BlockSpec((_TILE, 1), lambda i: (i, 0)),
            pl.BlockSpec((_D, 1), lambda i: (0, 0)),
            pl.BlockSpec((1, 1), lambda i: (0, 0)),
            pl.BlockSpec((1, _HID), lambda i: (0, 0)),
            pl.BlockSpec((1, _HID), lambda i: (0, 0)),
            pl.BlockSpec((_HID, _HID), lambda i: (0, 0)),
        ],
        out_specs=[
            pl.BlockSpec((_TILE, _HID), lambda i: (i, 0)),
            pl.BlockSpec((1, 4, _D), lambda i: (i, 0, 0)),
            pl.BlockSpec((1, 2, _HID), lambda i: (i, 0, 0)),
            pl.BlockSpec((_TILE, 1), lambda i: (i, 0)),
        ],
        out_shape=[
            jax.ShapeDtypeStruct((_B, _HID), _F32),
            jax.ShapeDtypeStruct((_NT, 4, _D), _F32),
            jax.ShapeDtypeStruct((_NT, 2, _HID), _F32),
            jax.ShapeDtypeStruct((_B, 1), _F32),
        ],
        compiler_params=_cp(),
    )(x0, Y1p, xw0, v, c, gb, bb, W2)


def _k2_body(x0_ref, y2_ref, xw0_ref, xw1_ref, v_ref, c_ref, gb_ref, bb_ref,
             wb_ref, zh_ref, mom_ref, xw2_ref):
    x0 = x0_ref[...]
    xw0 = xw0_ref[...]
    xw1 = xw1_ref[...]
    dd = jnp.dot(x0, v_ref[...], preferred_element_type=_F32)
    xw2 = dd[:, 0:1] * (1.0 + xw0) + dd[:, 1:2] * xw1 + c_ref[0, 0]
    h2 = jnp.maximum(y2_ref[...] * gb_ref[...] + bb_ref[...], 0.0)
    zh = jnp.dot(h2, wb_ref[...], preferred_element_type=_F32)
    zh_ref[...] = zh
    x2 = x0 * x0
    m4 = jnp.sum(x0 * xw2, 0, keepdims=True)
    p14 = jnp.sum(x2 * xw2, 0, keepdims=True)
    p24 = jnp.sum(x2 * (xw0 * xw2), 0, keepdims=True)
    p34 = jnp.sum(x2 * (xw1 * xw2), 0, keepdims=True)
    p44 = jnp.sum(x2 * (xw2 * xw2), 0, keepdims=True)
    mom_ref[...] = jnp.concatenate([m4, p14, p24, p34, p44], 0)[None]
    xw2_ref[...] = xw2


def _k2(x0, Y2p, xw0, xw1, v, c, gb, bb, wb):
    return pl.pallas_call(
        _k2_body,
        grid=(_NT,),
        in_specs=[
            pl.BlockSpec((_TILE, _D), lambda i: (i, 0)),
            pl.BlockSpec((_TILE, _HID), lambda i: (i, 0)),
            pl.BlockSpec((_TILE, 1), lambda i: (i, 0)),
            pl.BlockSpec((_TILE, 1), lambda i: (i, 0)),
            pl.BlockSpec((_D, 2), lambda i: (0, 0)),
            pl.BlockSpec((1, 1), lambda i: (0, 0)),
            pl.BlockSpec((1, _HID), lambda i: (0, 0)),
            pl.BlockSpec((1, _HID), lambda i: (0, 0)),
            pl.BlockSpec((_HID, 1), lambda i: (0, 0)),
        ],
        out_specs=[
            pl.BlockSpec((_TILE, 1), lambda i: (i, 0)),
            pl.BlockSpec((1, 5, _D), lambda i: (i, 0, 0)),
            pl.BlockSpec((_TILE, 1), lambda i: (i, 0)),
        ],
        out_shape=[
            jax.ShapeDtypeStruct((_B, 1), _F32),
            jax.ShapeDtypeStruct((_NT, 5, _D), _F32),
            jax.ShapeDtypeStruct((_B, 1), _F32),
        ],
        compiler_params=_cp(),
    )(x0, Y2p, xw0, xw1, v, c, gb, bb, wb)


def _k3_body(x0_ref, xw0_ref, xw1_ref, xw2_ref, zh_ref, f_ref, c_ref,
             z_ref, zs_ref):
    x0 = x0_ref[...]
    f = jnp.dot(x0, f_ref[...], preferred_element_type=_F32)
    z = (f[:, 0:1] * (1.0 + xw0_ref[...]) + f[:, 1:2] * xw1_ref[...]
         + f[:, 2:3] * xw2_ref[...] + zh_ref[...] + c_ref[0, 0])
    z_ref[...] = z
    zs_ref[...] = jnp.concatenate(
        [jnp.broadcast_to(jnp.sum(z), (1, 1, 128)),
         jnp.broadcast_to(jnp.sum(z * z), (1, 1, 128))], axis=1)


def _k3(x0, xw0, xw1, xw2, zh, fmat, c):
    return pl.pallas_call(
        _k3_body,
        grid=(_NT,),
        in_specs=[
            pl.BlockSpec((_TILE, _D), lambda i: (i, 0)),
            pl.BlockSpec((_TILE, 1), lambda i: (i, 0)),
            pl.BlockSpec((_TILE, 1), lambda i: (i, 0)),
            pl.BlockSpec((_TILE, 1), lambda i: (i, 0)),
            pl.BlockSpec((_TILE, 1), lambda i: (i, 0)),
            pl.BlockSpec((_D, 3), lambda i: (0, 0)),
            pl.BlockSpec((1, 1), lambda i: (0, 0)),
        ],
        out_specs=[
            pl.BlockSpec((_TILE, 1), lambda i: (i, 0)),
            pl.BlockSpec((1, 2, 128), lambda i: (i, 0, 0)),
        ],
        out_shape=[
            jax.ShapeDtypeStruct((_B, 1), _F32),
            jax.ShapeDtypeStruct((_NT, 2, 128), _F32),
        ],
        compiler_params=_cp(),
    )(x0, xw0, xw1, xw2, zh, fmat, c)


def _k4_body(z_ref, ab_ref, out_ref):
    out_ref[...] = z_ref[...] * ab_ref[0, 0] + ab_ref[0, 1]


def _k4(z, ab):
    return pl.pallas_call(
        _k4_body,
        grid=(_NT,),
        in_specs=[
            pl.BlockSpec((_TILE, 1), lambda i: (i, 0)),
            pl.BlockSpec((1, 2), lambda i: (0, 0)),
        ],
        out_specs=pl.BlockSpec((_TILE, 1), lambda i: (i, 0)),
        out_shape=jax.ShapeDtypeStruct((_B, 1), _F32),
        compiler_params=_cp(),
    )(z, ab)


def _dense_forward(x0, cross_w, cross_b, cross_g, cross_beta, W1, b1, g1,
                   beta1, W2, b2, g2, beta2, Wout, bout, gout, betaout):
    Bf = float(_B)
    wa = Wout[:_D, 0]
    wb = Wout[_D:, :]

    Y1p, mom0, y1s, xw0 = _k0(x0, W1, cross_w[0].reshape(_D, 1))
    mom0 = jnp.sum(mom0, 0) / Bf
    M1, M2, P11, P12, P22 = (mom0[0], mom0[1], mom0[2], mom0[3], mom0[4])
    y1s = jnp.sum(y1s, 0) / Bf

    t0p = cross_b[0]
    m0 = M1 + M2 + t0p
    Q0 = P11 + 2.0 * P12 + P22
    v0 = Q0 + 2.0 * t0p * (m0 - t0p) + t0p * t0p - m0 * m0
    G0 = cross_g[0] / jnp.sqrt(v0 + _EPS)
    t1 = (t0p - m0) * G0 + cross_beta[0]
    m1bn = y1s[0] + b1
    v1bn = y1s[1] - y1s[0] * y1s[0]
    Gb1 = g1 / jnp.sqrt(v1bn + _EPS)
    Bb1 = beta1 + (b1 - m1bn) * Gb1

    c1 = jnp.dot(t1, cross_w[1]).reshape(1, 1)
    Y2p, mom1, y2s, xw1 = _k1(x0, Y1p, xw0, (G0 * cross_w[1]).reshape(_D, 1),
                              c1, Gb1.reshape(1, _HID), Bb1.reshape(1, _HID),
                              W2)
    mom1 = jnp.sum(mom1, 0) / Bf
    M3, P13, P23, P33 = (mom1[0], mom1[1], mom1[2], mom1[3])
    y2s = jnp.sum(y2s, 0) / Bf

    t1p = t1 + cross_b[1]
    m1 = G0 * (M1 + M2) + M3 + t1p
    Q1 = (G0 * G0 * (P11 + 2.0 * P12 + P22) + 2.0 * G0 * (P13 + P23) + P33)
    v1 = Q1 + 2.0 * t1p * (m1 - t1p) + t1p * t1p - m1 * m1
    G1 = cross_g[1] / jnp.sqrt(v1 + _EPS)
    t2 = (t1p - m1) * G1 + cross_beta[1]
    m2bn = y2s[0] + b2
    v2bn = y2s[1] - y2s[0] * y2s[0]
    Gb2 = g2 / jnp.sqrt(v2bn + _EPS)
    Bb2 = beta2 + (b2 - m2bn) * Gb2

    c2 = jnp.dot(t2, cross_w[2]).reshape(1, 1)
    v2mat = jnp.stack([G0 * G1 * cross_w[2], G1 * cross_w[2]], axis=1)
    zh, mom2, xw2 = _k2(x0, Y2p, xw0, xw1, v2mat, c2, Gb2.reshape(1, _HID),
                        Bb2.reshape(1, _HID), wb)
    mom2 = jnp.sum(mom2, 0) / Bf
    M4, P14, P24, P34, P44 = (mom2[0], mom2[1], mom2[2], mom2[3], mom2[4])

    t2p = t2 + cross_b[2]
    a_, b_ = G0 * G1, G1
    m2 = a_ * (M1 + M2) + b_ * M3 + M4 + t2p
    Q2 = (a_ * a_ * (P11 + 2.0 * P12 + P22) + 2.0 * a_ * b_ * (P13 + P23)
          + b_ * b_ * P33 + 2.0 * a_ * (P14 + P24) + 2.0 * b_ * P34 + P44)
    v2 = Q2 + 2.0 * t2p * (m2 - t2p) + t2p * t2p - m2 * m2
    G2 = cross_g[2] / jnp.sqrt(v2 + _EPS)
    t3 = (t2p - m2) * G2 + cross_beta[2]

    c3 = (jnp.dot(t3, wa) + bout[0]).reshape(1, 1)
    fmat = jnp.stack([G0 * G1 * G2 * wa, G1 * G2 * wa, G2 * wa], axis=1)
    z, zs = _k3(x0, xw0, xw1, xw2, zh, fmat, c3)
    zs = jnp.sum(zs, 0)
    mz = zs[0, 0] / Bf
    vz = zs[1, 0] / Bf - mz * mz
    az = gout[0] / jnp.sqrt(vz + _EPS)
    ab = jnp.stack([az, betaout[0] - mz * az]).reshape(1, 2)
    return _k4(z, ab).reshape(_B)


def kernel(question_embedding, query_from_type, user_dense, gender, platform,
           age, location, user_idx, recent_clicked_note_idxs, note_embedding,
           note_dense, note_type, taxonomy1_id, taxonomy2_id, taxonomy3_id,
           note_idx, emb_gender, emb_platform, emb_age, emb_location,
           emb_user_idx, emb_note_type, emb_tax1, emb_tax2, emb_tax3,
           emb_note_idx, cross_w, cross_b, cross_g, cross_beta,
           W1, b1, g1, beta1, W2, b2, g2, beta2, Wout, bout, gout, betaout):
    i32 = jnp.int32
    r2d = lambda x: x.astype(i32).reshape(_B // 128, 128)
    eye16 = jnp.eye(16, dtype=_F32)
    (hist, noterow, qftoh, g_, p_, a_, loc_, u_, nt_, t1_, t2_, t3_) = \
        _sc_gather_all(
            emb_note_idx, recent_clicked_note_idxs.astype(i32).reshape(-1),
            r2d(note_idx), r2d(query_from_type), r2d(gender), r2d(platform),
            r2d(age), r2d(location), r2d(user_idx), r2d(note_type),
            r2d(taxonomy1_id), r2d(taxonomy2_id), r2d(taxonomy3_id),
            eye16, emb_gender, emb_platform, emb_age, emb_location,
            emb_user_idx, emb_note_type, emb_tax1, emb_tax2, emb_tax3)
    combined = jnp.concatenate(
        [question_embedding, qftoh, user_dense, g_, p_, a_, loc_, u_, hist,
         note_dense, nt_, t1_, t2_, t3_, noterow, note_embedding], axis=1)
    return _dense_forward(combined, cross_w, cross_b, cross_g, cross_beta,
                          W1, b1, g1, beta1, W2, b2, g2, beta2,
                          Wout, bout, gout, betaout)


# speedup vs baseline: 6.8578x; 1.1663x over previous
import functools

import jax
import jax.numpy as jnp
from jax import lax
from jax.experimental import pallas as pl
from jax.experimental.pallas import tpu as pltpu
from jax.experimental.pallas import tpu_sc as plsc

_B = 16384
_H = 200
_D = 1910
_HID = 256
_EPS = 1e-5
_NW = 32
_RW = _B // _NW
_GROUP = 8
_F32 = jnp.float32


_SCH = 64
_SW = (32, 16, 8, 8, 16, 32, 32, 8, 16, 32, 64)


def _sc_body(tab, hist2d, idxs, eye16, tg, tp, ta, tl, tu, tnt, tt1, tt2, tt3,
             o_hist, o_note, o_qft, o_g, o_p, o_a, o_l, o_u, o_nt, o_t1,
             o_t2, o_t3,
             sidx, b_note, b_qft, b_g, b_p, b_a, b_l, b_u, b_nt, b_t1, b_t2,
             b_t3, hidxA1, hidxA2, hidxB1, hidxB2, hrowsA, hrowsB, haccA,
             haccB, gsem, ssem, hsemA, hsemB, stsemA, stsemB):
    wid = lax.axis_index("s") * 2 + lax.axis_index("c")

    tabs = (tab, eye16, tg, tp, ta, tl, tu, tnt, tt1, tt2, tt3)
    bufs = (b_note, b_qft, b_g, b_p, b_a, b_l, b_u, b_nt, b_t1, b_t2, b_t3)
    outs = (o_note, o_qft, o_g, o_p, o_a, o_l, o_u, o_nt, o_t1, o_t2, o_t3)

    def stage_fire(g, hidx1, hidx2, hrows, hsem):
        r0 = wid * _RW + g * _GROUP
        pltpu.sync_copy(hist2d.at[pl.ds(r0, _GROUP), pl.ds(0, 128)], hidx1)
        pltpu.sync_copy(hist2d.at[pl.ds(r0, _GROUP), pl.ds(128, 72)], hidx2)
        for r in range(_GROUP):
            pltpu.async_copy(tab.at[hidx1.at[r]],
                             hrows.at[pl.ds(r * _H, 128)], hsem)
            pltpu.async_copy(tab.at[hidx2.at[r]],
                             hrows.at[pl.ds(r * _H + 128, 72)], hsem)

    def drain_gathers(hrows, hsem):
        pltpu.make_async_copy(tab.at[pl.ds(0, _GROUP * _H)], hrows,
                              hsem).wait()

    def accumulate(hrows, hacc):
        def row_acc(r, _):
            base = r * _H
            zero = jnp.zeros((16,), _F32)

            def step(k, carry):
                a0, a1, b0, b1 = carry
                o = base + k * 8
                for u in range(8):
                    lo = hrows[o + u, pl.ds(0, 16)]
                    hi = hrows[o + u, pl.ds(16, 16)]
                    if u % 2 == 0:
                        a0 = a0 + lo
                        a1 = a1 + hi
                    else:
                        b0 = b0 + lo
                        b1 = b1 + hi
                return (a0, a1, b0, b1)

            a0, a1, b0, b1 = lax.fori_loop(0, _H // 8, step,
                                           (zero, zero, zero, zero))
            hacc[r, pl.ds(0, 16)] = (a0 + b0) * (1.0 / _H)
            hacc[r, pl.ds(16, 16)] = (a1 + b1) * (1.0 / _H)
            return 0

        lax.fori_loop(0, _GROUP, row_acc, 0)

    def fire_store(g, hacc, stsem):
        r0 = wid * _RW + g * _GROUP
        pltpu.async_copy(hacc, o_hist.at[pl.ds(r0, _GROUP)], stsem)

    def drain_store(hacc, stsem):
        pltpu.make_async_copy(hacc, o_hist.at[pl.ds(0, _GROUP)], stsem).wait()

    stage_fire(0, hidxA1, hidxA2, hrowsA, hsemA)
    stage_fire(1, hidxB1, hidxB2, hrowsB, hsemB)

    def small_chunk(c, _):
        base = wid * _RW + c * _SCH
        pltpu.sync_copy(idxs.at[:, pl.ds(base, _SCH)], sidx)

        @pl.when(c > 0)
        def _():
            for t in range(11):
                pltpu.make_async_copy(bufs[t], outs[t].at[pl.ds(0, _SCH)],
                                      ssem).wait()

        cps = [pltpu.async_copy(tabs[t].at[sidx.at[t]], bufs[t], gsem)
               for t in range(11)]
        for cp in cps:
            cp.wait()
        for t in range(11):
            pltpu.async_copy(bufs[t], outs[t].at[pl.ds(base, _SCH)], ssem)
        return 0

    lax.fori_loop(0, _RW // _SCH, small_chunk, 0)
    for t in range(11):
        pltpu.make_async_copy(bufs[t], outs[t].at[pl.ds(0, _SCH)], ssem).wait()

    n_pairs = _RW // _GROUP // 2

    def pair(i, _):
        drain_gathers(hrowsA, hsemA)

        @pl.when(i > 0)
        def _():
            drain_store(haccA, stsemA)

        accumulate(hrowsA, haccA)
        fire_store(2 * i, haccA, stsemA)

        @pl.when(i < n_pairs - 1)
        def _():
            stage_fire(2 * i + 2, hidxA1, hidxA2, hrowsA, hsemA)

        drain_gathers(hrowsB, hsemB)

        @pl.when(i > 0)
        def _():
            drain_store(haccB, stsemB)

        accumulate(hrowsB, haccB)
        fire_store(2 * i + 1, haccB, stsemB)

        @pl.when(i < n_pairs - 1)
        def _():
            stage_fire(2 * i + 3, hidxB1, hidxB2, hrowsB, hsemB)

        return 0

    lax.fori_loop(0, n_pairs, pair, 0)
    drain_store(haccA, stsemA)
    drain_store(haccB, stsemB)


def _sc_gather_all(tab, hist2d, idxs, eye16, tg, tp, ta, tl, tu, tnt,
                   tt1, tt2, tt3):
    mesh = plsc.VectorSubcoreMesh(core_axis_name="c", subcore_axis_name="s")
    out_type = [
        jax.ShapeDtypeStruct((_B, 32), _F32),
        jax.ShapeDtypeStruct((_B, 32), _F32),
        jax.ShapeDtypeStruct((_B, 16), _F32),
        jax.ShapeDtypeStruct((_B, 8), _F32),
        jax.ShapeDtypeStruct((_B, 8), _F32),
        jax.ShapeDtypeStruct((_B, 16), _F32),
        jax.ShapeDtypeStruct((_B, 32), _F32),
        jax.ShapeDtypeStruct((_B, 32), _F32),
        jax.ShapeDtypeStruct((_B, 8), _F32),
        jax.ShapeDtypeStruct((_B, 16), _F32),
        jax.ShapeDtypeStruct((_B, 32), _F32),
        jax.ShapeDtypeStruct((_B, 64), _F32),
    ]
    scratch_types = (
        [pltpu.VMEM((11, _SCH), jnp.int32)]
        + [pltpu.VMEM((_SCH, w), _F32) for w in _SW]
        + [pltpu.VMEM((_GROUP, 128), jnp.int32),
           pltpu.VMEM((_GROUP, 72), jnp.int32),
           pltpu.VMEM((_GROUP, 128), jnp.int32),
           pltpu.VMEM((_GROUP, 72), jnp.int32),
           pltpu.VMEM((_GROUP * _H, 32), _F32),
           pltpu.VMEM((_GROUP * _H, 32), _F32),
           pltpu.VMEM((_GROUP, 32), _F32),
           pltpu.VMEM((_GROUP, 32), _F32)]
        + [pltpu.SemaphoreType.DMA] * 6
    )
    fn = functools.partial(
        pl.kernel, mesh=mesh, out_type=out_type,
        scratch_types=scratch_types,
        compiler_params=pltpu.CompilerParams(use_tc_tiling_on_sc=False),
    )(_sc_body)
    return fn(tab, hist2d, idxs, eye16, tg, tp, ta, tl, tu, tnt, tt1, tt2,
              tt3)


_TILE = 512
_NT = _B // _TILE


def _cp():
    return pltpu.CompilerParams(dimension_semantics=("arbitrary",))


def _k0_body(x0_ref, w1_ref, w0_ref, y1_ref, mom_ref, ys_ref, xw0_ref):
    x0 = x0_ref[...]
    xw0 = jnp.dot(x0, w0_ref[...], preferred_element_type=_F32)
    y1 = jnp.dot(x0, w1_ref[...], preferred_element_type=_F32)
    y1_ref[...] = y1
    x2 = x0 * x0
    m1 = jnp.sum(x0, 0, keepdims=True)
    m2 = jnp.sum(x0 * xw0, 0, keepdims=True)
    p11 = jnp.sum(x2, 0, keepdims=True)
    p12 = jnp.sum(x2 * xw0, 0, keepdims=True)
    p22 = jnp.sum(x2 * (xw0 * xw0), 0, keepdims=True)
    mom_ref[...] = jnp.concatenate([m1, m2, p11, p12, p22], 0)[None]
    ys_ref[...] = jnp.concatenate([jnp.sum(y1, 0, keepdims=True),
                                   jnp.sum(y1 * y1, 0, keepdims=True)],
                                  0)[None]
    xw0_ref[...] = xw0


def _k0(x0, W1, w0):
    return pl.pallas_call(
        _k0_body,
        grid=(_NT,),
        in_specs=[
            pl.BlockSpec((_TILE, _D), lambda i: (i, 0)),
            pl.BlockSpec((_D, _HID), lambda i: (0, 0)),
            pl.BlockSpec((_D, 1), lambda i: (0, 0)),
        ],
        out_specs=[
            pl.BlockSpec((_TILE, _HID), lambda i: (i, 0)),
            pl.BlockSpec((1, 5, _D), lambda i: (i, 0, 0)),
            pl.BlockSpec((1, 2, _HID), lambda i: (i, 0, 0)),
            pl.BlockSpec((_TILE, 1), lambda i: (i, 0)),
        ],
        out_shape=[
            jax.ShapeDtypeStruct((_B, _HID), _F32),
            jax.ShapeDtypeStruct((_NT, 5, _D), _F32),
            jax.ShapeDtypeStruct((_NT, 2, _HID), _F32),
            jax.ShapeDtypeStruct((_B, 1), _F32),
        ],
        compiler_params=_cp(),
    )(x0, W1, w0)


def _k1_body(x0_ref, y1_ref, xw0_ref, v_ref, c_ref, gb_ref, bb_ref, w2_ref,
             y2_ref, mom_ref, ys_ref, xw1_ref):
    x0 = x0_ref[...]
    xw0 = xw0_ref[...]
    e = jnp.dot(x0, v_ref[...], preferred_element_type=_F32)
    xw1 = e * (1.0 + xw0) + c_ref[0, 0]
    h1 = jnp.maximum(y1_ref[...] * gb_ref[...] + bb_ref[...], 0.0)
    y2 = jnp.dot(h1, w2_ref[...], preferred_element_type=_F32)
    y2_ref[...] = y2
    x2 = x0 * x0
    m3 = jnp.sum(x0 * xw1, 0, keepdims=True)
    p13 = jnp.sum(x2 * xw1, 0, keepdims=True)
    p23 = jnp.sum(x2 * (xw0 * xw1), 0, keepdims=True)
    p33 = jnp.sum(x2 * (xw1 * xw1), 0, keepdims=True)
    mom_ref[...] = jnp.concatenate([m3, p13, p23, p33], 0)[None]
    ys_ref[...] = jnp.concatenate([jnp.sum(y2, 0, keepdims=True),
                                   jnp.sum(y2 * y2, 0, keepdims=True)],
                                  0)[None]
    xw1_ref[...] = xw1


def _k1(x0, Y1p, xw0, v, c, gb, bb, W2):
    return pl.pallas_call(
        _k1_body,
        grid=(_NT,),
        in_specs=[
            pl.BlockSpec((_TILE, _D), lambda i: (i, 0)),
            pl.BlockSpec((_TILE, _HID), lambda i: (i, 0)),
            pl.BlockSpec((_TILE, 1), lambda i: (i, 0)),
            pl.BlockSpec((_D, 1), lambda i: (0, 0)),
            pl.BlockSpec((1, 1), lambda i: (0, 0)),
            pl.BlockSpec((1, _HID), lambda i: (0, 0)),
            pl.BlockSpec((1, _HID), lambda i: (0, 0)),
            pl.BlockSpec((_HID, _HID), lambda i: (0, 0)),
        ],
        out_specs=[
            pl.BlockSpec((_TILE, _HID), lambda i: (i, 0)),
            pl.BlockSpec((1, 4, _D), lambda i: (i, 0, 0)),
            pl.BlockSpec((1, 2, _HID), lambda i: (i, 0, 0)),
            pl.BlockSpec((_TILE, 1), lambda i: (i, 0)),
        ],
        out_shape=[
            jax.ShapeDtypeStruct((_B, _HID), _F32),
            jax.ShapeDtypeStruct((_NT, 4, _D), _F32),
            jax.ShapeDtypeStruct((_NT, 2, _HID), _F32),
            jax.ShapeDtypeStruct((_B, 1), _F32),
        ],
        compiler_params=_cp(),
    )(x0, Y1p, xw0, v, c, gb, bb, W2)


def _k2_body(x0_ref, y2_ref, xw0_ref, xw1_ref, v_ref, c_ref, gb_ref, bb_ref,
             wb_ref, zh_ref, mom_ref, xw2_ref):
    x0 = x0_ref[...]
    xw0 = xw0_ref[...]
    xw1 = xw1_ref[...]
    dd = jnp.dot(x0, v_ref[...], preferred_element_type=_F32)
    xw2 = dd[:, 0:1] * (1.0 + xw0) + dd[:, 1:2] * xw1 + c_ref[0, 0]
    h2 = jnp.maximum(y2_ref[...] * gb_ref[...] + bb_ref[...], 0.0)
    zh = jnp.dot(h2, wb_ref[...], preferred_element_type=_F32)
    zh_ref[...] = zh
    x2 = x0 * x0
    m4 = jnp.sum(x0 * xw2, 0, keepdims=True)
    p14 = jnp.sum(x2 * xw2, 0, keepdims=True)
    p24 = jnp.sum(x2 * (xw0 * xw2), 0, keepdims=True)
    p34 = jnp.sum(x2 * (xw1 * xw2), 0, keepdims=True)
    p44 = jnp.sum(x2 * (xw2 * xw2), 0, keepdims=True)
    mom_ref[...] = jnp.concatenate([m4, p14, p24, p34, p44], 0)[None]
    xw2_ref[...] = xw2


def _k2(x0, Y2p, xw0, xw1, v, c, gb, bb, wb):
    return pl.pallas_call(
        _k2_body,
        grid=(_NT,),
        in_specs=[
            pl.BlockSpec((_TILE, _D), lambda i: (i, 0)),
            pl.BlockSpec((_TILE, _HID), lambda i: (i, 0)),
            pl.BlockSpec((_TILE, 1), lambda i: (i, 0)),
            pl.BlockSpec((_TILE, 1), lambda i: (i, 0)),
            pl.BlockSpec((_D, 2), lambda i: (0, 0)),
            pl.BlockSpec((1, 1), lambda i: (0, 0)),
            pl.BlockSpec((1, _HID), lambda i: (0, 0)),
            pl.BlockSpec((1, _HID), lambda i: (0, 0)),
            pl.BlockSpec((_HID, 1), lambda i: (0, 0)),
        ],
        out_specs=[
            pl.BlockSpec((_TILE, 1), lambda i: (i, 0)),
            pl.BlockSpec((1, 5, _D), lambda i: (i, 0, 0)),
            pl.BlockSpec((_TILE, 1), lambda i: (i, 0)),
        ],
        out_shape=[
            jax.ShapeDtypeStruct((_B, 1), _F32),
            jax.ShapeDtypeStruct((_NT, 5, _D), _F32),
            jax.ShapeDtypeStruct((_B, 1), _F32),
        ],
        compiler_params=_cp(),
    )(x0, Y2p, xw0, xw1, v, c, gb, bb, wb)


def _k3_body(x0_ref, xw0_ref, xw1_ref, xw2_ref, zh_ref, f_ref, c_ref,
             z_ref, zs_ref):
    x0 = x0_ref[...]
    f = jnp.dot(x0, f_ref[...], preferred_element_type=_F32)
    z = (f[:, 0:1] * (1.0 + xw0_ref[...]) + f[:, 1:2] * xw1_ref[...]
         + f[:, 2:3] * xw2_ref[...] + zh_ref[...] + c_ref[0, 0])
    z_ref[...] = z
    zs_ref[...] = jnp.concatenate(
        [jnp.broadcast_to(jnp.sum(z), (1, 1, 128)),
         jnp.broadcast_to(jnp.sum(z * z), (1, 1, 128))], axis=1)


def _k3(x0, xw0, xw1, xw2, zh, fmat, c):
    return pl.pallas_call(
        _k3_body,
        grid=(_NT,),
        in_specs=[
            pl.BlockSpec((_TILE, _D), lambda i: (i, 0)),
            pl.BlockSpec((_TILE, 1), lambda i: (i, 0)),
            pl.BlockSpec((_TILE, 1), lambda i: (i, 0)),
            pl.BlockSpec((_TILE, 1), lambda i: (i, 0)),
            pl.BlockSpec((_TILE, 1), lambda i: (i, 0)),
            pl.BlockSpec((_D, 3), lambda i: (0, 0)),
            pl.BlockSpec((1, 1), lambda i: (0, 0)),
        ],
        out_specs=[
            pl.BlockSpec((_TILE, 1), lambda i: (i, 0)),
            pl.BlockSpec((1, 2, 128), lambda i: (i, 0, 0)),
        ],
        out_shape=[
            jax.ShapeDtypeStruct((_B, 1), _F32),
            jax.ShapeDtypeStruct((_NT, 2, 128), _F32),
        ],
        compiler_params=_cp(),
    )(x0, xw0, xw1, xw2, zh, fmat, c)


def _k4_body(z_ref, ab_ref, out_ref):
    out_ref[...] = z_ref[...] * ab_ref[0, 0] + ab_ref[0, 1]


def _k4(z, ab):
    return pl.pallas_call(
        _k4_body,
        grid=(_NT,),
        in_specs=[
            pl.BlockSpec((_TILE, 1), lambda i: (i, 0)),
            pl.BlockSpec((1, 2), lambda i: (0, 0)),
        ],
        out_specs=pl.BlockSpec((_TILE, 1), lambda i: (i, 0)),
        out_shape=jax.ShapeDtypeStruct((_B, 1), _F32),
        compiler_params=_cp(),
    )(z, ab)


def _dense_forward(x0, cross_w, cross_b, cross_g, cross_beta, W1, b1, g1,
                   beta1, W2, b2, g2, beta2, Wout, bout, gout, betaout):
    Bf = float(_B)
    wa = Wout[:_D, 0]
    wb = Wout[_D:, :]

    Y1p, mom0, y1s, xw0 = _k0(x0, W1, cross_w[0].reshape(_D, 1))
    mom0 = jnp.sum(mom0, 0) / Bf
    M1, M2, P11, P12, P22 = (mom0[0], mom0[1], mom0[2], mom0[3], mom0[4])
    y1s = jnp.sum(y1s, 0) / Bf

    t0p = cross_b[0]
    m0 = M1 + M2 + t0p
    Q0 = P11 + 2.0 * P12 + P22
    v0 = Q0 + 2.0 * t0p * (m0 - t0p) + t0p * t0p - m0 * m0
    G0 = cross_g[0] / jnp.sqrt(v0 + _EPS)
    t1 = (t0p - m0) * G0 + cross_beta[0]
    m1bn = y1s[0] + b1
    v1bn = y1s[1] - y1s[0] * y1s[0]
    Gb1 = g1 / jnp.sqrt(v1bn + _EPS)
    Bb1 = beta1 + (b1 - m1bn) * Gb1

    c1 = jnp.dot(t1, cross_w[1]).reshape(1, 1)
    Y2p, mom1, y2s, xw1 = _k1(x0, Y1p, xw0, (G0 * cross_w[1]).reshape(_D, 1),
                              c1, Gb1.reshape(1, _HID), Bb1.reshape(1, _HID),
                              W2)
    mom1 = jnp.sum(mom1, 0) / Bf
    M3, P13, P23, P33 = (mom1[0], mom1[1], mom1[2], mom1[3])
    y2s = jnp.sum(y2s, 0) / Bf

    t1p = t1 + cross_b[1]
    m1 = G0 * (M1 + M2) + M3 + t1p
    Q1 = (G0 * G0 * (P11 + 2.0 * P12 + P22) + 2.0 * G0 * (P13 + P23) + P33)
    v1 = Q1 + 2.0 * t1p * (m1 - t1p) + t1p * t1p - m1 * m1
    G1 = cross_g[1] / jnp.sqrt(v1 + _EPS)
    t2 = (t1p - m1) * G1 + cross_beta[1]
    m2bn = y2s[0] + b2
    v2bn = y2s[1] - y2s[0] * y2s[0]
    Gb2 = g2 / jnp.sqrt(v2bn + _EPS)
    Bb2 = beta2 + (b2 - m2bn) * Gb2

    c2 = jnp.dot(t2, cross_w[2]).reshape(1, 1)
    v2mat = jnp.stack([G0 * G1 * cross_w[2], G1 * cross_w[2]], axis=1)
    zh, mom2, xw2 = _k2(x0, Y2p, xw0, xw1, v2mat, c2, Gb2.reshape(1, _HID),
                        Bb2.reshape(1, _HID), wb)
    mom2 = jnp.sum(mom2, 0) / Bf
    M4, P14, P24, P34, P44 = (mom2[0], mom2[1], mom2[2], mom2[3], mom2[4])

    t2p = t2 + cross_b[2]
    a_, b_ = G0 * G1, G1
    m2 = a_ * (M1 + M2) + b_ * M3 + M4 + t2p
    Q2 = (a_ * a_ * (P11 + 2.0 * P12 + P22) + 2.0 * a_ * b_ * (P13 + P23)
          + b_ * b_ * P33 + 2.0 * a_ * (P14 + P24) + 2.0 * b_ * P34 + P44)
    v2 = Q2 + 2.0 * t2p * (m2 - t2p) + t2p * t2p - m2 * m2
    G2 = cross_g[2] / jnp.sqrt(v2 + _EPS)
    t3 = (t2p - m2) * G2 + cross_beta[2]

    c3 = (jnp.dot(t3, wa) + bout[0]).reshape(1, 1)
    fmat = jnp.stack([G0 * G1 * G2 * wa, G1 * G2 * wa, G2 * wa], axis=1)
    z, zs = _k3(x0, xw0, xw1, xw2, zh, fmat, c3)
    zs = jnp.sum(zs, 0)
    mz = zs[0, 0] / Bf
    vz = zs[1, 0] / Bf - mz * mz
    az = gout[0] / jnp.sqrt(vz + _EPS)
    ab = jnp.stack([az, betaout[0] - mz * az]).reshape(1, 2)
    return _k4(z, ab).reshape(_B)


def kernel(question_embedding, query_from_type, user_dense, gender, platform,
           age, location, user_idx, recent_clicked_note_idxs, note_embedding,
           note_dense, note_type, taxonomy1_id, taxonomy2_id, taxonomy3_id,
           note_idx, emb_gender, emb_platform, emb_age, emb_location,
           emb_user_idx, emb_note_type, emb_tax1, emb_tax2, emb_tax3,
           emb_note_idx, cross_w, cross_b, cross_g, cross_beta,
           W1, b1, g1, beta1, W2, b2, g2, beta2, Wout, bout, gout, betaout):
    i32 = jnp.int32
    eye16 = jnp.eye(16, dtype=_F32)
    idxs = jnp.stack([note_idx, query_from_type, gender, platform, age,
                      location, user_idx, note_type, taxonomy1_id,
                      taxonomy2_id, taxonomy3_id]).astype(i32)
    (hist, noterow, qftoh, g_, p_, a_, loc_, u_, nt_, t1_, t2_, t3_) = \
        _sc_gather_all(
            emb_note_idx, recent_clicked_note_idxs.astype(i32), idxs,
            eye16, emb_gender, emb_platform, emb_age, emb_location,
            emb_user_idx, emb_note_type, emb_tax1, emb_tax2, emb_tax3)
    combined = jnp.concatenate(
        [question_embedding, qftoh, user_dense, g_, p_, a_, loc_, u_, hist,
         note_dense, nt_, t1_, t2_, t3_, noterow, note_embedding], axis=1)
    return _dense_forward(combined, cross_w, cross_b, cross_g, cross_beta,
                          W1, b1, g1, beta1, W2, b2, g2, beta2,
                          Wout, bout, gout, betaout)


# speedup vs baseline: 7.4112x; 1.0807x over previous
import functools

import numpy as np

import jax
import jax.numpy as jnp
from jax import lax
from jax.experimental import pallas as pl
from jax.experimental.pallas import tpu as pltpu
from jax.experimental.pallas import tpu_sc as plsc

_B = 16384
_H = 200
_D = 1910
_HID = 256
_EPS = 1e-5
_NW = 32
_RW = _B // _NW
_GROUP = 8
_F32 = jnp.float32

_PERM = np.concatenate([np.arange(0, 922), np.arange(954, 1110),
                        np.arange(1142, 1910), np.arange(922, 954),
                        np.arange(1110, 1142)]).astype(np.int32)


_SCH = 64
_SW = (16, 8, 8, 16, 32, 32, 8, 16, 32, 64)


def _sc_small_body(idxs, eye16, tg, tp, ta, tl, tu, tnt, tt1, tt2, tt3,
                   o_qft, o_g, o_p, o_a, o_l, o_u, o_nt, o_t1, o_t2, o_t3,
                   sidx, b_qft, b_g, b_p, b_a, b_l, b_u, b_nt, b_t1, b_t2,
                   b_t3, gsem, ssem):
    wid = lax.axis_index("s") * 2 + lax.axis_index("c")
    tabs = (eye16, tg, tp, ta, tl, tu, tnt, tt1, tt2, tt3)
    bufs = (b_qft, b_g, b_p, b_a, b_l, b_u, b_nt, b_t1, b_t2, b_t3)
    outs = (o_qft, o_g, o_p, o_a, o_l, o_u, o_nt, o_t1, o_t2, o_t3)

    def small_chunk(c, _):
        base = wid * _RW + c * _SCH
        pltpu.sync_copy(idxs.at[:, pl.ds(base, _SCH)], sidx)

        @pl.when(c > 0)
        def _():
            for t in range(10):
                pltpu.make_async_copy(bufs[t], outs[t].at[pl.ds(0, _SCH)],
                                      ssem).wait()

        cps = [pltpu.async_copy(tabs[t].at[sidx.at[t]], bufs[t], gsem)
               for t in range(10)]
        for cp in cps:
            cp.wait()
        for t in range(10):
            pltpu.async_copy(bufs[t], outs[t].at[pl.ds(base, _SCH)], ssem)
        return 0

    lax.fori_loop(0, _RW // _SCH, small_chunk, 0)
    for t in range(10):
        pltpu.make_async_copy(bufs[t], outs[t].at[pl.ds(0, _SCH)], ssem).wait()


def _sc_small(idxs, eye16, tg, tp, ta, tl, tu, tnt, tt1, tt2, tt3):
    mesh = plsc.VectorSubcoreMesh(core_axis_name="c", subcore_axis_name="s")
    out_type = [jax.ShapeDtypeStruct((_B, w), _F32) for w in _SW]
    scratch_types = (
        [pltpu.VMEM((10, _SCH), jnp.int32)]
        + [pltpu.VMEM((_SCH, w), _F32) for w in _SW]
        + [pltpu.SemaphoreType.DMA] * 2
    )
    fn = functools.partial(
        pl.kernel, mesh=mesh, out_type=out_type,
        scratch_types=scratch_types,
        compiler_params=pltpu.CompilerParams(use_tc_tiling_on_sc=False),
    )(_sc_small_body)
    return fn(idxs, eye16, tg, tp, ta, tl, tu, tnt, tt1, tt2, tt3)


def _sc_hist_body(tab, hist2d, note1d, o_hist, o_note,
                  nidx, nrows, hidxA1, hidxA2, hidxB1, hidxB2, hrowsA,
                  hrowsB, haccA, haccB, gsem, ssem, hsemA, hsemB, stsemA,
                  stsemB):
    wid = lax.axis_index("s") * 2 + lax.axis_index("c")

    def stage_fire(g, hidx1, hidx2, hrows, hsem):
        r0 = wid * _RW + g * _GROUP
        pltpu.sync_copy(hist2d.at[pl.ds(r0, _GROUP), pl.ds(0, 128)], hidx1)
        pltpu.sync_copy(hist2d.at[pl.ds(r0, _GROUP), pl.ds(128, 72)], hidx2)
        for r in range(_GROUP):
            pltpu.async_copy(tab.at[hidx1.at[r]],
                             hrows.at[pl.ds(r * _H, 128)], hsem)
            pltpu.async_copy(tab.at[hidx2.at[r]],
                             hrows.at[pl.ds(r * _H + 128, 72)], hsem)

    def drain_gathers(hrows, hsem):
        pltpu.make_async_copy(tab.at[pl.ds(0, _GROUP * _H)], hrows,
                              hsem).wait()

    def accumulate(hrows, hacc):
        def row_acc(r, _):
            base = r * _H
            zero = jnp.zeros((16,), _F32)

            def step(k, carry):
                a0, a1, b0, b1 = carry
                o = base + k * 8
                for u in range(8):
                    lo = hrows[o + u, pl.ds(0, 16)]
                    hi = hrows[o + u, pl.ds(16, 16)]
                    if u % 2 == 0:
                        a0 = a0 + lo
                        a1 = a1 + hi
                    else:
                        b0 = b0 + lo
                        b1 = b1 + hi
                return (a0, a1, b0, b1)

            a0, a1, b0, b1 = lax.fori_loop(0, _H // 8, step,
                                           (zero, zero, zero, zero))
            hacc[r, pl.ds(0, 16)] = (a0 + b0) * (1.0 / _H)
            hacc[r, pl.ds(16, 16)] = (a1 + b1) * (1.0 / _H)
            return 0

        lax.fori_loop(0, _GROUP, row_acc, 0)

    def fire_store(g, hacc, stsem):
        r0 = wid * _RW + g * _GROUP
        pltpu.async_copy(hacc, o_hist.at[pl.ds(r0, _GROUP)], stsem)

    def drain_store(hacc, stsem):
        pltpu.make_async_copy(hacc, o_hist.at[pl.ds(0, _GROUP)], stsem).wait()

    stage_fire(0, hidxA1, hidxA2, hrowsA, hsemA)
    stage_fire(1, hidxB1, hidxB2, hrowsB, hsemB)

    def note_chunk(c, _):
        base = wid * _RW + c * 128
        pltpu.sync_copy(note1d.at[pl.ds(base, 128)], nidx)

        @pl.when(c > 0)
        def _():
            pltpu.make_async_copy(nrows, o_note.at[pl.ds(0, 128)],
                                  ssem).wait()

        pltpu.async_copy(tab.at[nidx], nrows, gsem).wait()
        pltpu.async_copy(nrows, o_note.at[pl.ds(base, 128)], ssem)
        return 0

    lax.fori_loop(0, _RW // 128, note_chunk, 0)
    pltpu.make_async_copy(nrows, o_note.at[pl.ds(0, 128)], ssem).wait()

    n_pairs = _RW // _GROUP // 2

    def pair(i, _):
        drain_gathers(hrowsA, hsemA)

        @pl.when(i > 0)
        def _():
            drain_store(haccA, stsemA)

        accumulate(hrowsA, haccA)
        fire_store(2 * i, haccA, stsemA)

        @pl.when(i < n_pairs - 1)
        def _():
            stage_fire(2 * i + 2, hidxA1, hidxA2, hrowsA, hsemA)

        drain_gathers(hrowsB, hsemB)

        @pl.when(i > 0)
        def _():
            drain_store(haccB, stsemB)

        accumulate(hrowsB, haccB)
        fire_store(2 * i + 1, haccB, stsemB)

        @pl.when(i < n_pairs - 1)
        def _():
            stage_fire(2 * i + 3, hidxB1, hidxB2, hrowsB, hsemB)

        return 0

    lax.fori_loop(0, n_pairs, pair, 0)
    drain_store(haccA, stsemA)
    drain_store(haccB, stsemB)


def _sc_hist(tab, hist2d, note1d):
    mesh = plsc.VectorSubcoreMesh(core_axis_name="c", subcore_axis_name="s")
    out_type = [
        jax.ShapeDtypeStruct((_B, 32), _F32),
        jax.ShapeDtypeStruct((_B, 32), _F32),
    ]
    scratch_types = (
        [pltpu.VMEM((128,), jnp.int32),
         pltpu.VMEM((128, 32), _F32),
         pltpu.VMEM((_GROUP, 128), jnp.int32),
         pltpu.VMEM((_GROUP, 72), jnp.int32),
         pltpu.VMEM((_GROUP, 128), jnp.int32),
         pltpu.VMEM((_GROUP, 72), jnp.int32),
         pltpu.VMEM((_GROUP * _H, 32), _F32),
         pltpu.VMEM((_GROUP * _H, 32), _F32),
         pltpu.VMEM((_GROUP, 32), _F32),
         pltpu.VMEM((_GROUP, 32), _F32)]
        + [pltpu.SemaphoreType.DMA] * 6
    )
    fn = functools.partial(
        pl.kernel, mesh=mesh, out_type=out_type,
        scratch_types=scratch_types,
        compiler_params=pltpu.CompilerParams(use_tc_tiling_on_sc=False),
    )(_sc_hist_body)
    return fn(tab, hist2d, note1d)


_TILE = 512
_NT = _B // _TILE


def _cp():
    return pltpu.CompilerParams(dimension_semantics=("arbitrary",))


def _k0_body(c2_ref, hist_ref, note_ref, w1_ref, w0_ref, x0_ref, y1_ref,
             mom_ref, ys_ref, xw0_ref):
    x0 = jnp.concatenate([c2_ref[...], hist_ref[...], note_ref[...]], axis=1)
    x0_ref[...] = x0
    xw0 = jnp.dot(x0, w0_ref[...], preferred_element_type=_F32)
    y1 = jnp.dot(x0, w1_ref[...], preferred_element_type=_F32)
    y1_ref[...] = y1
    x2 = x0 * x0
    m1 = jnp.sum(x0, 0, keepdims=True)
    m2 = jnp.sum(x0 * xw0, 0, keepdims=True)
    p11 = jnp.sum(x2, 0, keepdims=True)
    p12 = jnp.sum(x2 * xw0, 0, keepdims=True)
    p22 = jnp.sum(x2 * (xw0 * xw0), 0, keepdims=True)
    mom_ref[...] = jnp.concatenate([m1, m2, p11, p12, p22], 0)[None]
    ys_ref[...] = jnp.concatenate([jnp.sum(y1, 0, keepdims=True),
                                   jnp.sum(y1 * y1, 0, keepdims=True)],
                                  0)[None]
    xw0_ref[...] = xw0


_DC2 = _D - 64


def _k0(c2, hist, note, W1, w0):
    return pl.pallas_call(
        _k0_body,
        grid=(_NT,),
        in_specs=[
            pl.BlockSpec((_TILE, _DC2), lambda i: (i, 0)),
            pl.BlockSpec((_TILE, 32), lambda i: (i, 0)),
            pl.BlockSpec((_TILE, 32), lambda i: (i, 0)),
            pl.BlockSpec((_D, _HID), lambda i: (0, 0)),
            pl.BlockSpec((_D, 1), lambda i: (0, 0)),
        ],
        out_specs=[
            pl.BlockSpec((_TILE, _D), lambda i: (i, 0)),
            pl.BlockSpec((_TILE, _HID), lambda i: (i, 0)),
            pl.BlockSpec((1, 5, _D), lambda i: (i, 0, 0)),
            pl.BlockSpec((1, 2, _HID), lambda i: (i, 0, 0)),
            pl.BlockSpec((_TILE, 1), lambda i: (i, 0)),
        ],
        out_shape=[
            jax.ShapeDtypeStruct((_B, _D), _F32),
            jax.ShapeDtypeStruct((_B, _HID), _F32),
            jax.ShapeDtypeStruct((_NT, 5, _D), _F32),
            jax.ShapeDtypeStruct((_NT, 2, _HID), _F32),
            jax.ShapeDtypeStruct((_B, 1), _F32),
        ],
        compiler_params=_cp(),
    )(c2, hist, note, W1, w0)


def _k1_body(x0_ref, y1_ref, xw0_ref, v_ref, c_ref, gb_ref, bb_ref, w2_ref,
             y2_ref, mom_ref, ys_ref, xw1_ref):
    x0 = x0_ref[...]
    xw0 = xw0_ref[...]
    e = jnp.dot(x0, v_ref[...], preferred_element_type=_F32)
    xw1 = e * (1.0 + xw0) + c_ref[0, 0]
    h1 = jnp.maximum(y1_ref[...] * gb_ref[...] + bb_ref[...], 0.0)
    y2 = jnp.dot(h1, w2_ref[...], preferred_element_type=_F32)
    y2_ref[...] = y2
    x2 = x0 * x0
    m3 = jnp.sum(x0 * xw1, 0, keepdims=True)
    p13 = jnp.sum(x2 * xw1, 0, keepdims=True)
    p23 = jnp.sum(x2 * (xw0 * xw1), 0, keepdims=True)
    p33 = jnp.sum(x2 * (xw1 * xw1), 0, keepdims=True)
    mom_ref[...] = jnp.concatenate([m3, p13, p23, p33], 0)[None]
    ys_ref[...] = jnp.concatenate([jnp.sum(y2, 0, keepdims=True),
                                   jnp.sum(y2 * y2, 0, keepdims=True)],
                                  0)[None]
    xw1_ref[...] = xw1


def _k1(x0, Y1p, xw0, v, c, gb, bb, W2):
    return pl.pallas_call(
        _k1_body,
        grid=(_NT,),
        in_specs=[
            pl.BlockSpec((_TILE, _D), lambda i: (i, 0)),
            pl.BlockSpec((_TILE, _HID), lambda i: (i, 0)),
            pl.BlockSpec((_TILE, 1), lambda i: (i, 0)),
            pl.BlockSpec((_D, 1), lambda i: (0, 0)),
            pl.BlockSpec((1, 1), lambda i: (0, 0)),
            pl.BlockSpec((1, _HID), lambda i: (0, 0)),
            pl.BlockSpec((1, _HID), lambda i: (0, 0)),
            pl.BlockSpec((_HID, _HID), lambda i: (0, 0)),
        ],
        out_specs=[
            pl.BlockSpec((_TILE, _HID), lambda i: (i, 0)),
            pl.BlockSpec((1, 4, _D), lambda i: (i, 0, 0)),
            pl.BlockSpec((1, 2, _HID), lambda i: (i, 0, 0)),
            pl.BlockSpec((_TILE, 1), lambda i: (i, 0)),
        ],
        out_shape=[
            jax.ShapeDtypeStruct((_B, _HID), _F32),
            jax.ShapeDtypeStruct((_NT, 4, _D), _F32),
            jax.ShapeDtypeStruct((_NT, 2, _HID), _F32),
            jax.ShapeDtypeStruct((_B, 1), _F32),
        ],
        compiler_params=_cp(),
    )(x0, Y1p, xw0, v, c, gb, bb, W2)


def _k2_body(x0_ref, y2_ref, xw0_ref, xw1_ref, v_ref, c_ref, gb_ref, bb_ref,
             wb_ref, zh_ref, mom_ref, xw2_ref):
    x0 = x0_ref[...]
    xw0 = xw0_ref[...]
    xw1 = xw1_ref[...]
    dd = jnp.dot(x0, v_ref[...], preferred_element_type=_F32)
    xw2 = dd[:, 0:1] * (1.0 + xw0) + dd[:, 1:2] * xw1 + c_ref[0, 0]
    h2 = jnp.maximum(y2_ref[...] * gb_ref[...] + bb_ref[...], 0.0)
    zh = jnp.dot(h2, wb_ref[...], preferred_element_type=_F32)
    zh_ref[...] = zh
    x2 = x0 * x0
    m4 = jnp.sum(x0 * xw2, 0, keepdims=True)
    p14 = jnp.sum(x2 * xw2, 0, keepdims=True)
    p24 = jnp.sum(x2 * (xw0 * xw2), 0, keepdims=True)
    p34 = jnp.sum(x2 * (xw1 * xw2), 0, keepdims=True)
    p44 = jnp.sum(x2 * (xw2 * xw2), 0, keepdims=True)
    mom_ref[...] = jnp.concatenate([m4, p14, p24, p34, p44], 0)[None]
    xw2_ref[...] = xw2


def _k2(x0, Y2p, xw0, xw1, v, c, gb, bb, wb):
    return pl.pallas_call(
        _k2_body,
        grid=(_NT,),
        in_specs=[
            pl.BlockSpec((_TILE, _D), lambda i: (i, 0)),
            pl.BlockSpec((_TILE, _HID), lambda i: (i, 0)),
            pl.BlockSpec((_TILE, 1), lambda i: (i, 0)),
            pl.BlockSpec((_TILE, 1), lambda i: (i, 0)),
            pl.BlockSpec((_D, 2), lambda i: (0, 0)),
            pl.BlockSpec((1, 1), lambda i: (0, 0)),
            pl.BlockSpec((1, _HID), lambda i: (0, 0)),
            pl.BlockSpec((1, _HID), lambda i: (0, 0)),
            pl.BlockSpec((_HID, 1), lambda i: (0, 0)),
        ],
        out_specs=[
            pl.BlockSpec((_TILE, 1), lambda i: (i, 0)),
            pl.BlockSpec((1, 5, _D), lambda i: (i, 0, 0)),
            pl.BlockSpec((_TILE, 1), lambda i: (i, 0)),
        ],
        out_shape=[
            jax.ShapeDtypeStruct((_B, 1), _F32),
            jax.ShapeDtypeStruct((_NT, 5, _D), _F32),
            jax.ShapeDtypeStruct((_B, 1), _F32),
        ],
        compiler_params=_cp(),
    )(x0, Y2p, xw0, xw1, v, c, gb, bb, wb)


def _k3_body(x0_ref, xw0_ref, xw1_ref, xw2_ref, zh_ref, f_ref, c_ref,
             z_ref, zs_ref):
    x0 = x0_ref[...]
    f = jnp.dot(x0, f_ref[...], preferred_element_type=_F32)
    z = (f[:, 0:1] * (1.0 + xw0_ref[...]) + f[:, 1:2] * xw1_ref[...]
         + f[:, 2:3] * xw2_ref[...] + zh_ref[...] + c_ref[0, 0])
    z_ref[...] = z
    zs_ref[...] = jnp.concatenate(
        [jnp.broadcast_to(jnp.sum(z), (1, 1, 128)),
         jnp.broadcast_to(jnp.sum(z * z), (1, 1, 128))], axis=1)


def _k3(x0, xw0, xw1, xw2, zh, fmat, c):
    return pl.pallas_call(
        _k3_body,
        grid=(_NT,),
        in_specs=[
            pl.BlockSpec((_TILE, _D), lambda i: (i, 0)),
            pl.BlockSpec((_TILE, 1), lambda i: (i, 0)),
            pl.BlockSpec((_TILE, 1), lambda i: (i, 0)),
            pl.BlockSpec((_TILE, 1), lambda i: (i, 0)),
            pl.BlockSpec((_TILE, 1), lambda i: (i, 0)),
            pl.BlockSpec((_D, 3), lambda i: (0, 0)),
            pl.BlockSpec((1, 1), lambda i: (0, 0)),
        ],
        out_specs=[
            pl.BlockSpec((_TILE, 1), lambda i: (i, 0)),
            pl.BlockSpec((1, 2, 128), lambda i: (i, 0, 0)),
        ],
        out_shape=[
            jax.ShapeDtypeStruct((_B, 1), _F32),
            jax.ShapeDtypeStruct((_NT, 2, 128), _F32),
        ],
        compiler_params=_cp(),
    )(x0, xw0, xw1, xw2, zh, fmat, c)


def _k4_body(z_ref, ab_ref, out_ref):
    out_ref[...] = z_ref[...] * ab_ref[0, 0] + ab_ref[0, 1]


def _k4(z, ab):
    return pl.pallas_call(
        _k4_body,
        grid=(_NT,),
        in_specs=[
            pl.BlockSpec((_TILE, 1), lambda i: (i, 0)),
            pl.BlockSpec((1, 2), lambda i: (0, 0)),
        ],
        out_specs=pl.BlockSpec((_TILE, 1), lambda i: (i, 0)),
        out_shape=jax.ShapeDtypeStruct((_B, 1), _F32),
        compiler_params=_cp(),
    )(z, ab)


def _dense_forward(c2, hist, note, cross_w, cross_b, cross_g, cross_beta,
                   W1, b1, g1, beta1, W2, b2, g2, beta2, Wout, bout, gout,
                   betaout):
    Bf = float(_B)
    wa = Wout[:_D, 0]
    wb = Wout[_D:, :]

    x0, Y1p, mom0, y1s, xw0 = _k0(c2, hist, note, W1,
                                  cross_w[0].reshape(_D, 1))
    mom0 = jnp.sum(mom0, 0) / Bf
    M1, M2, P11, P12, P22 = (mom0[0], mom0[1], mom0[2], mom0[3], mom0[4])
    y1s = jnp.sum(y1s, 0) / Bf

    t0p = cross_b[0]
    m0 = M1 + M2 + t0p
    Q0 = P11 + 2.0 * P12 + P22
    v0 = Q0 + 2.0 * t0p * (m0 - t0p) + t0p * t0p - m0 * m0
    G0 = cross_g[0] / jnp.sqrt(v0 + _EPS)
    t1 = (t0p - m0) * G0 + cross_beta[0]
    m1bn = y1s[0] + b1
    v1bn = y1s[1] - y1s[0] * y1s[0]
    Gb1 = g1 / jnp.sqrt(v1bn + _EPS)
    Bb1 = beta1 + (b1 - m1bn) * Gb1

    c1 = jnp.dot(t1, cross_w[1]).reshape(1, 1)
    Y2p, mom1, y2s, xw1 = _k1(x0, Y1p, xw0, (G0 * cross_w[1]).reshape(_D, 1),
                              c1, Gb1.reshape(1, _HID), Bb1.reshape(1, _HID),
                              W2)
    mom1 = jnp.sum(mom1, 0) / Bf
    M3, P13, P23, P33 = (mom1[0], mom1[1], mom1[2], mom1[3])
    y2s = jnp.sum(y2s, 0) / Bf

    t1p = t1 + cross_b[1]
    m1 = G0 * (M1 + M2) + M3 + t1p
    Q1 = (G0 * G0 * (P11 + 2.0 * P12 + P22) + 2.0 * G0 * (P13 + P23) + P33)
    v1 = Q1 + 2.0 * t1p * (m1 - t1p) + t1p * t1p - m1 * m1
    G1 = cross_g[1] / jnp.sqrt(v1 + _EPS)
    t2 = (t1p - m1) * G1 + cross_beta[1]
    m2bn = y2s[0] + b2
    v2bn = y2s[1] - y2s[0] * y2s[0]
    Gb2 = g2 / jnp.sqrt(v2bn + _EPS)
    Bb2 = beta2 + (b2 - m2bn) * Gb2

    c2 = jnp.dot(t2, cross_w[2]).reshape(1, 1)
    v2mat = jnp.stack([G0 * G1 * cross_w[2], G1 * cross_w[2]], axis=1)
    zh, mom2, xw2 = _k2(x0, Y2p, xw0, xw1, v2mat, c2, Gb2.reshape(1, _HID),
                        Bb2.reshape(1, _HID), wb)
    mom2 = jnp.sum(mom2, 0) / Bf
    M4, P14, P24, P34, P44 = (mom2[0], mom2[1], mom2[2], mom2[3], mom2[4])

    t2p = t2 + cross_b[2]
    a_, b_ = G0 * G1, G1
    m2 = a_ * (M1 + M2) + b_ * M3 + M4 + t2p
    Q2 = (a_ * a_ * (P11 + 2.0 * P12 + P22) + 2.0 * a_ * b_ * (P13 + P23)
          + b_ * b_ * P33 + 2.0 * a_ * (P14 + P24) + 2.0 * b_ * P34 + P44)
    v2 = Q2 + 2.0 * t2p * (m2 - t2p) + t2p * t2p - m2 * m2
    G2 = cross_g[2] / jnp.sqrt(v2 + _EPS)
    t3 = (t2p - m2) * G2 + cross_beta[2]

    c3 = (jnp.dot(t3, wa) + bout[0]).reshape(1, 1)
    fmat = jnp.stack([G0 * G1 * G2 * wa, G1 * G2 * wa, G2 * wa], axis=1)
    z, zs = _k3(x0, xw0, xw1, xw2, zh, fmat, c3)
    zs = jnp.sum(zs, 0)
    mz = zs[0, 0] / Bf
    vz = zs[1, 0] / Bf - mz * mz
    az = gout[0] / jnp.sqrt(vz + _EPS)
    ab = jnp.stack([az, betaout[0] - mz * az]).reshape(1, 2)
    return _k4(z, ab).reshape(_B)


def kernel(question_embedding, query_from_type, user_dense, gender, platform,
           age, location, user_idx, recent_clicked_note_idxs, note_embedding,
           note_dense, note_type, taxonomy1_id, taxonomy2_id, taxonomy3_id,
           note_idx, emb_gender, emb_platform, emb_age, emb_location,
           emb_user_idx, emb_note_type, emb_tax1, emb_tax2, emb_tax3,
           emb_note_idx, cross_w, cross_b, cross_g, cross_beta,
           W1, b1, g1, beta1, W2, b2, g2, beta2, Wout, bout, gout, betaout):
    i32 = jnp.int32
    eye16 = jnp.eye(16, dtype=_F32)
    idxs = jnp.stack([query_from_type, gender, platform, age, location,
                      user_idx, note_type, taxonomy1_id, taxonomy2_id,
                      taxonomy3_id]).astype(i32)
    (qftoh, g_, p_, a_, loc_, u_, nt_, t1_, t2_, t3_) = _sc_small(
        idxs, eye16, emb_gender, emb_platform, emb_age, emb_location,
        emb_user_idx, emb_note_type, emb_tax1, emb_tax2, emb_tax3)
    hist, noterow = _sc_hist(emb_note_idx,
                             recent_clicked_note_idxs.astype(i32),
                             note_idx.astype(i32))
    c2 = jnp.concatenate(
        [question_embedding, qftoh, user_dense, g_, p_, a_, loc_, u_,
         note_dense, nt_, t1_, t2_, t3_, note_embedding], axis=1)
    return _dense_forward(c2, hist, noterow, cross_w[:, _PERM],
                          cross_b[:, _PERM], cross_g[:, _PERM],
                          cross_beta[:, _PERM], W1[_PERM], b1, g1, beta1,
                          W2, b2, g2, beta2,
                          jnp.concatenate([Wout[:_D][_PERM], Wout[_D:]], 0),
                          bout, gout, betaout)
